# Initial kernel scaffold; baseline (speedup 1.0000x reference)
import jax
import jax.numpy as jnp
from jax.experimental import pallas as pl


def kernel(x, W_in, b_in, W_ph, b_ph, W_gate, b_gate, W_pers, b_pers, W_thal, b_thal, W1, b1, W2, b2, W_out, b_out):
    raise NotImplementedError("write your pallas kernel here")


# speedup vs baseline: 3.9592x; 3.9592x over previous
import functools

import jax
import jax.numpy as jnp
from jax.experimental import pallas as pl

_HIDDEN = 768
_NUM_EXPERTS = 8
_D_FF = 1536
_NUM_CLASSES = 10
_H_PH = 32
_DELTA0 = 7.0
_TRAITS = (0.6, 0.6, 0.5, 0.5, 0.4)

_BT = 128


def _frontend_body(x_ref, Win_ref, bin_ref, Wph_ref, bph_ref, Wg_ref, bg_ref,
                   att_ref, gl0_ref, asum_ref):
    i = pl.program_id(0)
    x = x_ref[...]
    proj = jnp.dot(x, Win_ref[...], preferred_element_type=jnp.float32) + bin_ref[...]
    t = jnp.mean(proj, axis=-1, keepdims=True)
    k = (jax.lax.broadcasted_iota(jnp.int32, (_BT, _H_PH), 1) + 1).astype(jnp.float32)
    ph = _DELTA0 * k * t
    feats = jnp.concatenate([jnp.cos(ph), jnp.sin(ph)], axis=-1)
    tm = jnp.dot(feats, Wph_ref[...], preferred_element_type=jnp.float32) + bph_ref[...]
    enh = proj + tm
    cols = jax.lax.broadcasted_iota(jnp.int32, (_BT, _HIDDEN), 1)
    m = jnp.max(enh, axis=-1, keepdims=True)
    amax = jnp.min(jnp.where(enh == m, cols, _HIDDEN), axis=-1, keepdims=True)
    att = enh * (1.0 + (cols == amax).astype(jnp.float32))
    att_ref[...] = att
    gl0_ref[...] = jnp.dot(att, Wg_ref[...], preferred_element_type=jnp.float32) + bg_ref[...]

    @pl.when(i == 0)
    def _():
        asum_ref[...] = jnp.zeros_like(asum_ref)
    asum_ref[...] += jnp.sum(att, axis=0, keepdims=True)


def _gating_body(tr_ref, gl0_ref, asum_ref, Wth_ref, bth_ref, Wp_ref, bp_ref, gw_ref, B):
    pbias = jnp.dot(tr_ref[...], Wp_ref[...], preferred_element_type=jnp.float32) + bp_ref[...]
    thal = jnp.dot(asum_ref[...] * (1.0 / B), Wth_ref[...],
                   preferred_element_type=jnp.float32) + bth_ref[...]
    gl = gl0_ref[...] + pbias + thal
    cols = jax.lax.broadcasted_iota(jnp.int32, gl.shape, 1)
    m0 = jnp.max(gl, axis=-1, keepdims=True)
    c0 = jnp.min(jnp.where(gl == m0, cols, _NUM_EXPERTS), axis=-1, keepdims=True)
    glm = jnp.where(cols == c0, -1e30, gl)
    m1 = jnp.max(glm, axis=-1, keepdims=True)
    c1 = jnp.min(jnp.where(glm == m1, cols, _NUM_EXPERTS), axis=-1, keepdims=True)
    e1 = jnp.exp(m1 - m0)
    w0 = 1.0 / (1.0 + e1)
    w1 = e1 * w0
    gw_ref[...] = (w0 * (cols == c0) + w1 * (cols == c1)).astype(jnp.float32)


def _fold_body(W2_ref, b2_ref, Wout_ref, W2p_ref, c2_ref):
    W2p_ref[...] = jnp.dot(W2_ref[0], Wout_ref[...],
                           preferred_element_type=jnp.float32)[None]
    c2_ref[...] = jnp.dot(b2_ref[0], Wout_ref[...],
                          preferred_element_type=jnp.float32)[None]


def _expert_body(att_ref, W1_ref, b1_ref, W2p_ref, c2_ref, gw_ref, y_ref):
    e = pl.program_id(0)
    a = att_ref[...]
    h = jnp.maximum(jnp.dot(a, W1_ref[0], preferred_element_type=jnp.float32)
                    + b1_ref[0], 0.0)
    y = jnp.dot(h, W2p_ref[0], preferred_element_type=jnp.float32) + c2_ref[0]
    gw = gw_ref[...]
    onehot = (jax.lax.broadcasted_iota(jnp.int32, gw.shape, 1) == e)
    w = jnp.sum(gw * onehot, axis=-1, keepdims=True)
    y_ref[...] = (y * w)[None]


def _combine_body(y_ref, bout_ref, out_ref):
    out_ref[...] = jnp.sum(y_ref[...], axis=0) + bout_ref[...]


def kernel(x, W_in, b_in, W_ph, b_ph, W_gate, b_gate, W_pers, b_pers,
           W_thal, b_thal, W1, b1, W2, b2, W_out, b_out):
    B = x.shape[0]
    nt = B // _BT
    E = _NUM_EXPERTS
    f32 = jnp.float32

    r2 = lambda v: v.reshape(1, -1)
    NCP = 16
    W_out_p = jnp.pad(W_out, ((0, 0), (0, NCP - _NUM_CLASSES)))
    b_out_p = jnp.pad(b_out, (0, NCP - _NUM_CLASSES)).reshape(1, NCP)

    att, gl0, asum = pl.pallas_call(
        _frontend_body,
        grid=(nt,),
        in_specs=[
            pl.BlockSpec((_BT, x.shape[1]), lambda i: (i, 0)),
            pl.BlockSpec(W_in.shape, lambda i: (0, 0)),
            pl.BlockSpec((1, _HIDDEN), lambda i: (0, 0)),
            pl.BlockSpec(W_ph.shape, lambda i: (0, 0)),
            pl.BlockSpec((1, _HIDDEN), lambda i: (0, 0)),
            pl.BlockSpec(W_gate.shape, lambda i: (0, 0)),
            pl.BlockSpec((1, E), lambda i: (0, 0)),
        ],
        out_specs=[
            pl.BlockSpec((_BT, _HIDDEN), lambda i: (i, 0)),
            pl.BlockSpec((_BT, E), lambda i: (i, 0)),
            pl.BlockSpec((1, _HIDDEN), lambda i: (0, 0)),
        ],
        out_shape=[
            jax.ShapeDtypeStruct((B, _HIDDEN), f32),
            jax.ShapeDtypeStruct((B, E), f32),
            jax.ShapeDtypeStruct((1, _HIDDEN), f32),
        ],
    )(x, W_in, r2(b_in), W_ph, r2(b_ph), W_gate, r2(b_gate))

    traits = jnp.asarray(_TRAITS, dtype=f32).reshape(1, 5)
    gate_w = pl.pallas_call(
        functools.partial(_gating_body, B=float(B)),
        out_shape=jax.ShapeDtypeStruct((B, E), f32),
    )(traits, gl0, asum, W_thal, r2(b_thal), W_pers, r2(b_pers))

    W2p, c2 = pl.pallas_call(
        _fold_body,
        grid=(E,),
        in_specs=[
            pl.BlockSpec((1, _D_FF, _HIDDEN), lambda e: (e, 0, 0)),
            pl.BlockSpec((1, 1, _HIDDEN), lambda e: (e, 0, 0)),
            pl.BlockSpec((_HIDDEN, NCP), lambda e: (0, 0)),
        ],
        out_specs=[
            pl.BlockSpec((1, _D_FF, NCP), lambda e: (e, 0, 0)),
            pl.BlockSpec((1, 1, NCP), lambda e: (e, 0, 0)),
        ],
        out_shape=[
            jax.ShapeDtypeStruct((E, _D_FF, NCP), f32),
            jax.ShapeDtypeStruct((E, 1, NCP), f32),
        ],
    )(W2, b2.reshape(E, 1, _HIDDEN), W_out_p)

    Y = pl.pallas_call(
        _expert_body,
        grid=(E, nt),
        in_specs=[
            pl.BlockSpec((_BT, _HIDDEN), lambda e, t: (t, 0)),
            pl.BlockSpec((1, _HIDDEN, _D_FF), lambda e, t: (e, 0, 0)),
            pl.BlockSpec((1, 1, _D_FF), lambda e, t: (e, 0, 0)),
            pl.BlockSpec((1, _D_FF, NCP), lambda e, t: (e, 0, 0)),
            pl.BlockSpec((1, 1, NCP), lambda e, t: (e, 0, 0)),
            pl.BlockSpec((_BT, E), lambda e, t: (t, 0)),
        ],
        out_specs=pl.BlockSpec((1, _BT, NCP), lambda e, t: (e, t, 0)),
        out_shape=jax.ShapeDtypeStruct((E, B, NCP), f32),
    )(att, W1, b1.reshape(E, 1, _D_FF), W2p, c2, gate_w)

    out = pl.pallas_call(
        _combine_body,
        grid=(nt,),
        in_specs=[
            pl.BlockSpec((E, _BT, NCP), lambda t: (0, t, 0)),
            pl.BlockSpec((1, NCP), lambda t: (0, 0)),
        ],
        out_specs=pl.BlockSpec((_BT, NCP), lambda t: (t, 0)),
        out_shape=jax.ShapeDtypeStruct((B, NCP), f32),
    )(Y, b_out_p)

    return out[:, :_NUM_CLASSES]


# speedup vs baseline: 4.3439x; 1.0972x over previous
import functools

import jax
import jax.numpy as jnp
from jax import lax
from jax.experimental import pallas as pl
from jax.experimental.pallas import tpu as pltpu
from jax.experimental.pallas import tpu_sc as plsc

_HIDDEN = 768
_NUM_EXPERTS = 8
_D_FF = 1536
_NUM_CLASSES = 10
_H_PH = 32
_DELTA0 = 7.0
_TRAITS = (0.6, 0.6, 0.5, 0.5, 0.4)

_BT = 128
_NCP = 16
_NCW = 128
_NT_G = 24
_NP = _NT_G * _BT


def _frontend_body(x_ref, Win_ref, bin_ref, Wph_ref, bph_ref, Wg_ref, bg_ref,
                   att_ref, gl0_ref, asum_ref):
    i = pl.program_id(0)
    x = x_ref[...]
    proj = jnp.dot(x, Win_ref[...], preferred_element_type=jnp.float32) + bin_ref[...]
    t = jnp.mean(proj, axis=-1, keepdims=True)
    k = (jax.lax.broadcasted_iota(jnp.int32, (_BT, _H_PH), 1) + 1).astype(jnp.float32)
    ph = _DELTA0 * k * t
    feats = jnp.concatenate([jnp.cos(ph), jnp.sin(ph)], axis=-1)
    tm = jnp.dot(feats, Wph_ref[...], preferred_element_type=jnp.float32) + bph_ref[...]
    enh = proj + tm
    cols = jax.lax.broadcasted_iota(jnp.int32, (_BT, _HIDDEN), 1)
    m = jnp.max(enh, axis=-1, keepdims=True)
    amax = jnp.min(jnp.where(enh == m, cols, _HIDDEN), axis=-1, keepdims=True)
    att = enh * (1.0 + (cols == amax).astype(jnp.float32))
    att_ref[...] = att
    gl0_ref[...] = jnp.dot(att, Wg_ref[...], preferred_element_type=jnp.float32) + bg_ref[...]

    @pl.when(i == 0)
    def _():
        asum_ref[...] = jnp.zeros_like(asum_ref)
    asum_ref[...] += jnp.sum(att, axis=0, keepdims=True)


def _routing_body(tr_ref, gl0_ref, asum_ref, Wth_ref, bth_ref, Wp_ref, bp_ref,
                  p0_ref, p1_ref, w0_ref, w1_ref, te_ref, B):
    E = _NUM_EXPERTS
    f32 = jnp.float32
    pbias = jnp.dot(tr_ref[...], Wp_ref[...], preferred_element_type=f32) + bp_ref[...]
    thal = jnp.dot(asum_ref[...] * (1.0 / B), Wth_ref[...],
                   preferred_element_type=f32) + bth_ref[...]
    gl = gl0_ref[...] + pbias + thal
    Bi = gl.shape[0]
    cols = jax.lax.broadcasted_iota(jnp.int32, (Bi, E), 1)
    m0 = jnp.max(gl, axis=-1, keepdims=True)
    c0 = jnp.min(jnp.where(gl == m0, cols, E), axis=-1, keepdims=True)
    glm = jnp.where(cols == c0, -1e30, gl)
    m1 = jnp.max(glm, axis=-1, keepdims=True)
    c1 = jnp.min(jnp.where(glm == m1, cols, E), axis=-1, keepdims=True)
    e1 = jnp.exp(m1 - m0)
    ones16 = jnp.ones((1, _NCP), f32)
    w0_ref[...] = (1.0 / (1.0 + e1)) * ones16
    w1_ref[...] = (e1 / (1.0 + e1)) * ones16

    oh0 = (cols == c0).astype(f32)
    oh1 = (cols == c1).astype(f32)
    r_io = jax.lax.broadcasted_iota(jnp.int32, (Bi, Bi), 0)
    c_io = jax.lax.broadcasted_iota(jnp.int32, (Bi, Bi), 1)
    tril = (c_io < r_io).astype(f32)
    cnt01 = jnp.concatenate([oh0, oh1], axis=-1)
    excl = jnp.dot(tril, cnt01, preferred_element_type=f32)
    excl0, excl1 = excl[:, :E], excl[:, E:]
    total0 = jnp.sum(oh0, axis=0, keepdims=True)
    total1 = jnp.sum(oh1, axis=0, keepdims=True)
    counts = total0 + total1
    pc = jnp.floor((counts + (_BT - 1)) * (1.0 / _BT)) * float(_BT)
    e_r = jax.lax.broadcasted_iota(jnp.int32, (E, E), 0)
    e_c = jax.lax.broadcasted_iota(jnp.int32, (E, E), 1)
    triE = (e_r < e_c).astype(f32)
    offs = jnp.dot(pc, triE, preferred_element_type=f32)

    rank0 = jnp.sum(excl0 * oh0, axis=-1, keepdims=True)
    rank1 = jnp.sum((total0 + excl1) * oh1, axis=-1, keepdims=True)
    off0 = jnp.sum(offs * oh0, axis=-1, keepdims=True)
    off1 = jnp.sum(offs * oh1, axis=-1, keepdims=True)
    p0_ref[...] = (off0 + rank0).astype(jnp.int32)
    p1_ref[...] = (off1 + rank1).astype(jnp.int32)

    t_io = jax.lax.broadcasted_iota(jnp.int32, (_NT_G, E), 0).astype(f32) * float(_BT)
    offs_b = offs + jnp.zeros((_NT_G, E), f32)
    te_ref[...] = (jnp.sum((offs_b <= t_io).astype(jnp.int32), axis=-1,
                           keepdims=True) - 1)


def _fold_body(W2_ref, b2_ref, Wout_ref, W2p_ref, c2_ref):
    W2p_ref[...] = jnp.dot(W2_ref[0], Wout_ref[...],
                           preferred_element_type=jnp.float32)[None]
    c2_ref[...] = jnp.dot(b2_ref[0], Wout_ref[...],
                          preferred_element_type=jnp.float32)[None]


def _grouped_body(te_ref, xs_ref, W1_ref, b1_ref, W2p_ref, c2_ref, y_ref):
    h = jnp.maximum(jnp.dot(xs_ref[...], W1_ref[0],
                            preferred_element_type=jnp.float32) + b1_ref[0], 0.0)
    y_ref[...] = jnp.dot(h, W2p_ref[0], preferred_element_type=jnp.float32) + c2_ref[0]


def _dispatch_body(att_hbm, p0_hbm, p1_hbm, xs_hbm, a_v, p0_v, p1_v, sem0, sem1):
    nc = 2
    wid = lax.axis_index("s") * nc + lax.axis_index("c")
    n = a_v.shape[0]
    base = wid * n
    pltpu.sync_copy(att_hbm.at[pl.ds(base, n)], a_v)
    pltpu.sync_copy(p0_hbm.at[pl.ds(base, n)], p0_v)
    pltpu.sync_copy(p1_hbm.at[pl.ds(base, n)], p1_v)
    cp0 = pltpu.async_copy(a_v, xs_hbm.at[p0_v], sem0)
    cp1 = pltpu.async_copy(a_v, xs_hbm.at[p1_v], sem1)
    cp0.wait()
    cp1.wait()


def _combine_body(y_hbm, p0_hbm, p1_hbm, w0_hbm, w1_hbm, bout_hbm, out_hbm,
                  y0_v, y1_v, p0_v, p1_v, w0_v, w1_v, bout_v, o_v, sem0, sem1):
    nc = 2
    wid = lax.axis_index("s") * nc + lax.axis_index("c")
    n = p0_v.shape[0]
    base = wid * n
    pltpu.sync_copy(p0_hbm.at[pl.ds(base, n)], p0_v)
    pltpu.sync_copy(p1_hbm.at[pl.ds(base, n)], p1_v)
    pltpu.sync_copy(w0_hbm.at[pl.ds(base, n)], w0_v)
    pltpu.sync_copy(w1_hbm.at[pl.ds(base, n)], w1_v)
    pltpu.sync_copy(bout_hbm, bout_v)
    cp0 = pltpu.async_copy(y_hbm.at[p0_v], y0_v, sem0)
    cp1 = pltpu.async_copy(y_hbm.at[p1_v], y1_v, sem1)
    cp0.wait()
    cp1.wait()
    b = bout_v[...]
    for r in range(n):
        o_v[r, :] = (w0_v[r, :] * y0_v[r, pl.ds(0, _NCP)]
                     + w1_v[r, :] * y1_v[r, pl.ds(0, _NCP)] + b)
    pltpu.sync_copy(o_v, out_hbm.at[pl.ds(base, n)])


def kernel(x, W_in, b_in, W_ph, b_ph, W_gate, b_gate, W_pers, b_pers,
           W_thal, b_thal, W1, b1, W2, b2, W_out, b_out):
    B = x.shape[0]
    nt = B // _BT
    E = _NUM_EXPERTS
    f32 = jnp.float32

    r2 = lambda v: v.reshape(1, -1)
    W_out_p = jnp.pad(W_out, ((0, 0), (0, _NCW - _NUM_CLASSES)))
    b_out_p = jnp.pad(b_out, (0, _NCP - _NUM_CLASSES))

    att, gl0, asum = pl.pallas_call(
        _frontend_body,
        grid=(nt,),
        in_specs=[
            pl.BlockSpec((_BT, x.shape[1]), lambda i: (i, 0)),
            pl.BlockSpec(W_in.shape, lambda i: (0, 0)),
            pl.BlockSpec((1, _HIDDEN), lambda i: (0, 0)),
            pl.BlockSpec(W_ph.shape, lambda i: (0, 0)),
            pl.BlockSpec((1, _HIDDEN), lambda i: (0, 0)),
            pl.BlockSpec(W_gate.shape, lambda i: (0, 0)),
            pl.BlockSpec((1, E), lambda i: (0, 0)),
        ],
        out_specs=[
            pl.BlockSpec((_BT, _HIDDEN), lambda i: (i, 0)),
            pl.BlockSpec((_BT, E), lambda i: (i, 0)),
            pl.BlockSpec((1, _HIDDEN), lambda i: (0, 0)),
        ],
        out_shape=[
            jax.ShapeDtypeStruct((B, _HIDDEN), f32),
            jax.ShapeDtypeStruct((B, E), f32),
            jax.ShapeDtypeStruct((1, _HIDDEN), f32),
        ],
    )(x, W_in, r2(b_in), W_ph, r2(b_ph), W_gate, r2(b_gate))

    traits = jnp.asarray(_TRAITS, dtype=f32).reshape(1, 5)
    p0, p1, w0, w1, te = pl.pallas_call(
        functools.partial(_routing_body, B=float(B)),
        out_shape=[
            jax.ShapeDtypeStruct((B, 1), jnp.int32),
            jax.ShapeDtypeStruct((B, 1), jnp.int32),
            jax.ShapeDtypeStruct((B, _NCP), f32),
            jax.ShapeDtypeStruct((B, _NCP), f32),
            jax.ShapeDtypeStruct((_NT_G, 1), jnp.int32),
        ],
    )(traits, gl0, asum, W_thal, r2(b_thal), W_pers, r2(b_pers))
    p0 = p0.reshape(B)
    p1 = p1.reshape(B)
    te = te.reshape(_NT_G)

    W2p, c2 = pl.pallas_call(
        _fold_body,
        grid=(E,),
        in_specs=[
            pl.BlockSpec((1, _D_FF, _HIDDEN), lambda e: (e, 0, 0)),
            pl.BlockSpec((1, 1, _HIDDEN), lambda e: (e, 0, 0)),
            pl.BlockSpec((_HIDDEN, _NCW), lambda e: (0, 0)),
        ],
        out_specs=[
            pl.BlockSpec((1, _D_FF, _NCW), lambda e: (e, 0, 0)),
            pl.BlockSpec((1, 1, _NCW), lambda e: (e, 0, 0)),
        ],
        out_shape=[
            jax.ShapeDtypeStruct((E, _D_FF, _NCW), f32),
            jax.ShapeDtypeStruct((E, 1, _NCW), f32),
        ],
    )(W2, b2.reshape(E, 1, _HIDDEN), W_out_p)

    mesh = plsc.VectorSubcoreMesh(core_axis_name="c", subcore_axis_name="s")
    npw = B // 32
    xs = pl.kernel(
        _dispatch_body,
        out_type=jax.ShapeDtypeStruct((_NP, _HIDDEN), f32),
        mesh=mesh,
        scratch_types=[
            pltpu.VMEM((npw, _HIDDEN), f32),
            pltpu.VMEM((npw,), jnp.int32),
            pltpu.VMEM((npw,), jnp.int32),
            pltpu.SemaphoreType.DMA,
            pltpu.SemaphoreType.DMA,
        ],
    )(att, p0, p1)

    y = pl.pallas_call(
        _grouped_body,
        grid_spec=pltpu.PrefetchScalarGridSpec(
            num_scalar_prefetch=1,
            grid=(_NT_G,),
            in_specs=[
                pl.BlockSpec((_BT, _HIDDEN), lambda t, te_r: (t, 0)),
                pl.BlockSpec((1, _HIDDEN, _D_FF), lambda t, te_r: (te_r[t], 0, 0)),
                pl.BlockSpec((1, 1, _D_FF), lambda t, te_r: (te_r[t], 0, 0)),
                pl.BlockSpec((1, _D_FF, _NCW), lambda t, te_r: (te_r[t], 0, 0)),
                pl.BlockSpec((1, 1, _NCW), lambda t, te_r: (te_r[t], 0, 0)),
            ],
            out_specs=pl.BlockSpec((_BT, _NCW), lambda t, te_r: (t, 0)),
        ),
        out_shape=jax.ShapeDtypeStruct((_NP, _NCW), f32),
    )(te, xs, W1, b1.reshape(E, 1, _D_FF), W2p, c2)

    out16 = pl.kernel(
        _combine_body,
        out_type=jax.ShapeDtypeStruct((B, _NCP), f32),
        mesh=mesh,
        scratch_types=[
            pltpu.VMEM((npw, _NCW), f32),
            pltpu.VMEM((npw, _NCW), f32),
            pltpu.VMEM((npw,), jnp.int32),
            pltpu.VMEM((npw,), jnp.int32),
            pltpu.VMEM((npw, _NCP), f32),
            pltpu.VMEM((npw, _NCP), f32),
            pltpu.VMEM((_NCP,), f32),
            pltpu.VMEM((npw, _NCP), f32),
            pltpu.SemaphoreType.DMA,
            pltpu.SemaphoreType.DMA,
        ],
    )(y, p0, p1, w0, w1, b_out_p)

    return out16[:, :_NUM_CLASSES]
